# Initial kernel scaffold; baseline (speedup 1.0000x reference)
import jax
import jax.numpy as jnp
from jax.experimental import pallas as pl


def kernel(ps_terms, ps_term_ids, his_news):
    raise NotImplementedError("write your pallas kernel here")


# speedup vs baseline: 18.8170x; 18.8170x over previous
import jax
import jax.numpy as jnp
from jax import lax
from jax.experimental import pallas as pl
from jax.experimental.pallas import tpu as pltpu

HIS = 50
K = 5
N = HIS * K
D = 128
SL = 100
DK = K * D


def _fuse_body(his_ref, ids_ref, terms_ref, out_ref):
    his = his_ref[0].astype(jnp.float32)
    idc = ids_ref[0]
    terms = terms_ref[0]

    jj = lax.broadcasted_iota(jnp.int32, (N, 1), 0)
    iota_h = lax.broadcasted_iota(jnp.int32, (N, HIS), 1)
    iota_s = lax.broadcasted_iota(jnp.int32, (N, SL), 1)
    iota_l = lax.broadcasted_iota(jnp.int32, (N, N), 1)

    hv = jnp.broadcast_to(his[:, None, :], (HIS, K, SL)).reshape(N, SL)

    sel = (idc == iota_s).astype(jnp.float32)
    v = jnp.sum(sel * hv, axis=1, keepdims=True)

    eq = v == jnp.transpose(v)
    fp = jnp.min(jnp.where(eq, iota_l, N), axis=1, keepdims=True)
    is_first = (fp == jj).astype(jnp.float32)

    lcmp = (iota_l <= fp).astype(jnp.float32)
    gid = jnp.dot(lcmp, is_first,
                  preferred_element_type=jnp.float32) - 1.0

    a_oh = (gid.astype(jnp.int32) == iota_h).astype(jnp.float32)
    b_oh = (idc == iota_h).astype(jnp.float32)

    counts = lax.dot_general(a_oh, b_oh, (((0,), (0,)), ((), ())),
                             preferred_element_type=jnp.float32)
    out_ref[0] = jnp.dot(counts, terms, preferred_element_type=jnp.float32,
                         precision=lax.Precision.HIGHEST)


def kernel(ps_terms, ps_term_ids, his_news):
    B = ps_terms.shape[0]
    terms = ps_terms.reshape(B, HIS, DK)
    ids = ps_term_ids.reshape(B, N, 1)

    out = pl.pallas_call(
        _fuse_body,
        grid=(B,),
        in_specs=[
            pl.BlockSpec((1, HIS, SL), lambda i: (i, 0, 0)),
            pl.BlockSpec((1, N, 1), lambda i: (i, 0, 0)),
            pl.BlockSpec((1, HIS, DK), lambda i: (i, 0, 0)),
        ],
        out_specs=pl.BlockSpec((1, HIS, DK), lambda i: (i, 0, 0)),
        out_shape=jax.ShapeDtypeStruct((B, HIS, DK), jnp.float32),
    )(his_news, ids, terms)
    return out.reshape(B, N, D)


# speedup vs baseline: 21.2900x; 1.1314x over previous
import jax
import jax.numpy as jnp
from jax import lax
from jax.experimental import pallas as pl
from jax.experimental.pallas import tpu as pltpu

HIS = 50
K = 5
N = HIS * K
D = 128
SL = 100
DK = K * D
BS = 4


def _fuse_body(his_ref, ids_ref, terms_ref, out_ref):
    jj = lax.broadcasted_iota(jnp.int32, (N, 1), 0)
    iota_h = lax.broadcasted_iota(jnp.int32, (N, HIS), 1)
    iota_s = lax.broadcasted_iota(jnp.int32, (N, SL), 1)
    iota_l = lax.broadcasted_iota(jnp.int32, (N, N), 1)
    expand = (jj // K == iota_h).astype(jnp.float32)

    for e in range(BS):
        his = his_ref[e]
        idc = ids_ref[e]
        terms = terms_ref[e]

        his_lo = (his & 0xFF).astype(jnp.float32)
        his_hi = (his >> 8).astype(jnp.float32)
        hv_lo = jnp.dot(expand, his_lo, preferred_element_type=jnp.float32)
        hv_hi = jnp.dot(expand, his_hi, preferred_element_type=jnp.float32)

        sel = (idc == iota_s).astype(jnp.float32)
        v = (256.0 * jnp.sum(sel * hv_hi, axis=1, keepdims=True)
             + jnp.sum(sel * hv_lo, axis=1, keepdims=True))

        eq = v == jnp.transpose(v)
        fp = jnp.min(jnp.where(eq, iota_l, N), axis=1, keepdims=True)
        is_first = (fp == jj).astype(jnp.float32)

        lcmp = (iota_l <= fp).astype(jnp.float32)
        gid = jnp.dot(lcmp, is_first,
                      preferred_element_type=jnp.float32) - 1.0

        a_oh = (gid.astype(jnp.int32) == iota_h).astype(jnp.float32)
        b_oh = (idc == iota_h).astype(jnp.float32)

        counts = lax.dot_general(a_oh, b_oh, (((0,), (0,)), ((), ())),
                                 preferred_element_type=jnp.float32)
        out_ref[e] = jnp.dot(counts, terms, preferred_element_type=jnp.float32)


def kernel(ps_terms, ps_term_ids, his_news):
    B = ps_terms.shape[0]
    terms = ps_terms.reshape(B, HIS, DK)
    ids = ps_term_ids.reshape(B, N, 1)

    out = pl.pallas_call(
        _fuse_body,
        grid=(B // BS,),
        in_specs=[
            pl.BlockSpec((BS, HIS, SL), lambda i: (i, 0, 0)),
            pl.BlockSpec((BS, N, 1), lambda i: (i, 0, 0)),
            pl.BlockSpec((BS, HIS, DK), lambda i: (i, 0, 0)),
        ],
        out_specs=pl.BlockSpec((BS, HIS, DK), lambda i: (i, 0, 0)),
        out_shape=jax.ShapeDtypeStruct((B, HIS, DK), jnp.float32),
    )(his_news, ids, terms)
    return out.reshape(B, N, D)


# speedup vs baseline: 39.4902x; 1.8549x over previous
import functools

import jax
import jax.numpy as jnp
from jax import lax
from jax.experimental import pallas as pl
from jax.experimental.pallas import tpu as pltpu
from jax.experimental.pallas import tpu_sc as plsc

HIS = 50
K = 5
N = HIS * K
D = 128
SL = 100
DK = K * D
VOCAB = 30522
TBL = 30528
GCOLS = 64
CLEN = HIS * GCOLS
NCHUNK = 16
BS = 4


def _lane_iota():
    return lax.broadcasted_iota(jnp.int32, (16,), 0)


def _take16(x, idx):
    dnums = lax.GatherDimensionNumbers(
        offset_dims=(), collapsed_slice_dims=(0,), start_index_map=(0,))
    return lax.gather(x, idx[:, None], dnums, (1,),
                      mode=lax.GatherScatterMode.PROMISE_IN_BOUNDS)


def _sc_body(his_hbm, ids_hbm, cnt_hbm, his_v, ids_v, vbuf, table, cflat):
    nc = 2
    wid = lax.axis_index("s") * nc + lax.axis_index("c")
    lanes = _lane_iota()
    zeros16 = jnp.zeros((16,), jnp.int32)
    zerosf = jnp.zeros((16,), jnp.float32)
    ones16 = jnp.ones((16,), jnp.float32)

    def _zt(k, _):
        table[pl.ds(k * 16, 16)] = zeros16
        return _
    lax.fori_loop(0, TBL // 16, _zt, 0)

    def _example(ex, _):
        i = wid * 8 + ex
        pltpu.sync_copy(his_hbm.at[i], his_v)
        pltpu.sync_copy(ids_hbm.at[i], ids_v)

        def _zc(k, _c):
            cflat[pl.ds(k * 16, 16)] = zerosf
            return _c
        lax.fori_loop(0, CLEN // 16, _zc, 0)

        for c in range(NCHUNK):
            jvec = c * 16 + lanes
            hvec = jnp.minimum(jvec // K, HIS - 1)
            tvec = ids_v[pl.ds(c * 16, 16)]
            addr = hvec * SL + tvec
            vbuf[pl.ds(c * 16, 16)] = plsc.load_gather(his_v, [addr])

        cnt = jnp.int32(0)
        for c in range(NCHUNK):
            jvec = c * 16 + lanes
            lanemask = jvec < N
            vvec = vbuf[pl.ds(c * 16, 16)]
            g16 = plsc.load_gather(table, [vvec])
            fpl = jnp.full((16,), 16, jnp.int32)
            for b in range(15, -1, -1):
                bv = _take16(vvec, jnp.full((16,), b, jnp.int32))
                fpl = jnp.where(vvec == bv, b, fpl)
            new_first = (fpl == lanes) & (g16 == 0) & lanemask
            rank = jnp.cumsum(new_first.astype(jnp.int32))
            gid_new = cnt + rank - 1
            gid16 = jnp.where(g16 > 0, g16 - 1, _take16(gid_new, fpl))
            plsc.store_scatter(table, [vvec], gid16 + 1, mask=lanemask)
            cnt = cnt + jnp.sum(new_first.astype(jnp.int32))
            cmask = lanemask & (gid16 < HIS)
            caddr = gid16 * GCOLS + ids_v[pl.ds(c * 16, 16)]
            plsc.addupdate_scatter(cflat, [caddr], ones16, mask=cmask)

        for c in range(NCHUNK):
            jvec = c * 16 + lanes
            vvec = vbuf[pl.ds(c * 16, 16)]
            plsc.store_scatter(table, [vvec], zeros16, mask=jvec < N)

        pltpu.sync_copy(cflat, cnt_hbm.at[i])
        return _
    lax.fori_loop(0, 8, _example, 0)


def _counts_kernel(his2d, ids2d):
    mesh = plsc.VectorSubcoreMesh(core_axis_name="c", subcore_axis_name="s")
    f = functools.partial(
        pl.kernel, mesh=mesh,
        compiler_params=pltpu.CompilerParams(needs_layout_passes=False),
        out_type=jax.ShapeDtypeStruct((256, CLEN), jnp.float32),
        scratch_types=[
            pltpu.VMEM((HIS * SL,), jnp.int32),
            pltpu.VMEM((256,), jnp.int32),
            pltpu.VMEM((256,), jnp.int32),
            pltpu.VMEM((TBL,), jnp.int32),
            pltpu.VMEM((CLEN,), jnp.float32),
        ],
    )(_sc_body)
    return f(his2d, ids2d)


def _tc_body(cnt_ref, terms_ref, out_ref):
    for e in range(BS):
        counts = cnt_ref[e][:, :HIS]
        out_ref[e] = jnp.dot(counts, terms_ref[e],
                             preferred_element_type=jnp.float32)


def kernel(ps_terms, ps_term_ids, his_news):
    B = ps_terms.shape[0]
    terms = ps_terms.reshape(B, HIS, DK)
    ids = ps_term_ids.reshape(B, N)
    ids_pad = jnp.pad(ids, ((0, 0), (0, 256 - N)))
    his2d = his_news.reshape(B, HIS * SL)

    counts = _counts_kernel(his2d, ids_pad).reshape(B, HIS, GCOLS)

    out = pl.pallas_call(
        _tc_body,
        grid=(B // BS,),
        in_specs=[
            pl.BlockSpec((BS, HIS, GCOLS), lambda i: (i, 0, 0)),
            pl.BlockSpec((BS, HIS, DK), lambda i: (i, 0, 0)),
        ],
        out_specs=pl.BlockSpec((BS, HIS, DK), lambda i: (i, 0, 0)),
        out_shape=jax.ShapeDtypeStruct((B, HIS, DK), jnp.float32),
    )(counts, terms)
    return out.reshape(B, N, D)


# speedup vs baseline: 43.7160x; 1.1070x over previous
import functools

import jax
import jax.numpy as jnp
from jax import lax
from jax.experimental import pallas as pl
from jax.experimental.pallas import tpu as pltpu
from jax.experimental.pallas import tpu_sc as plsc

HIS = 50
K = 5
N = HIS * K
D = 128
SL = 100
DK = K * D
VOCAB = 30522
TBL = 30528
GCOLS = 64
CLEN = HIS * GCOLS
NCHUNK = 16
BS = 4


def _lane_iota():
    return lax.broadcasted_iota(jnp.int32, (16,), 0)


def _take16(x, idx):
    dnums = lax.GatherDimensionNumbers(
        offset_dims=(), collapsed_slice_dims=(0,), start_index_map=(0,))
    return lax.gather(x, idx[:, None], dnums, (1,),
                      mode=lax.GatherScatterMode.PROMISE_IN_BOUNDS)


def _sc_body(his_hbm, ids_hbm, cnt_hbm, his_v, ids_v, vbuf, table, cflat):
    nc = 2
    wid = lax.axis_index("s") * nc + lax.axis_index("c")
    lanes = _lane_iota()
    zeros16 = jnp.zeros((16,), jnp.int32)
    zerosf = jnp.zeros((16,), jnp.float32)
    ones16 = jnp.ones((16,), jnp.float32)

    def _zt(k, _):
        table[pl.ds(k * 16, 16)] = zeros16
        return _
    lax.fori_loop(0, TBL // 16, _zt, 0)

    def _example(ex, _):
        i = wid * 8 + ex
        pltpu.sync_copy(his_hbm.at[i], his_v)
        pltpu.sync_copy(ids_hbm.at[i], ids_v)

        def _zc(k, _c):
            cflat[pl.ds(k * 16, 16)] = zerosf
            return _c
        lax.fori_loop(0, CLEN // 16, _zc, 0)

        for c in range(NCHUNK):
            jvec = c * 16 + lanes
            hvec = jnp.minimum(jvec // K, HIS - 1)
            tvec = ids_v[pl.ds(c * 16, 16)]
            addr = hvec * SL + tvec
            vbuf[pl.ds(c * 16, 16)] = plsc.load_gather(his_v, [addr])

        cnt = jnp.int32(0)
        for c in range(NCHUNK):
            jvec = c * 16 + lanes
            lanemask = jvec < N
            vvec = vbuf[pl.ds(c * 16, 16)]
            g16 = plsc.load_gather(table, [vvec])
            fpl = jnp.full((16,), 16, jnp.int32)
            for b in range(15, -1, -1):
                bv = _take16(vvec, jnp.full((16,), b, jnp.int32))
                fpl = jnp.where(vvec == bv, b, fpl)
            new_first = (fpl == lanes) & (g16 == 0) & lanemask
            rank = jnp.cumsum(new_first.astype(jnp.int32))
            gid_new = cnt + rank - 1
            gid16 = jnp.where(g16 > 0, g16 - 1, _take16(gid_new, fpl))
            plsc.store_scatter(table, [vvec], gid16 + 1, mask=lanemask)
            cnt = cnt + jnp.sum(new_first.astype(jnp.int32))
            cmask = lanemask & (gid16 < HIS)
            caddr = gid16 * GCOLS + ids_v[pl.ds(c * 16, 16)]
            plsc.addupdate_scatter(cflat, [caddr], ones16, mask=cmask)

        for c in range(NCHUNK):
            jvec = c * 16 + lanes
            vvec = vbuf[pl.ds(c * 16, 16)]
            plsc.store_scatter(table, [vvec], zeros16, mask=jvec < N)

        pltpu.sync_copy(cflat, cnt_hbm.at[i])
        return _
    lax.fori_loop(0, 8, _example, 0)


def _counts_kernel(his2d, ids2d):
    mesh = plsc.VectorSubcoreMesh(core_axis_name="c", subcore_axis_name="s")
    f = functools.partial(
        pl.kernel, mesh=mesh,
        compiler_params=pltpu.CompilerParams(needs_layout_passes=False),
        out_type=jax.ShapeDtypeStruct((256, CLEN), jnp.float32),
        scratch_types=[
            pltpu.VMEM((HIS * SL,), jnp.int32),
            pltpu.VMEM((256,), jnp.int32),
            pltpu.VMEM((256,), jnp.int32),
            pltpu.VMEM((TBL,), jnp.int32),
            pltpu.VMEM((CLEN,), jnp.float32),
        ],
    )(_sc_body)
    return f(his2d, ids2d)


def _tc_body(cnt_ref, terms_ref, out_ref):
    jrow = lax.broadcasted_iota(jnp.int32, (N, HIS), 0)
    gcol = lax.broadcasted_iota(jnp.int32, (N, HIS), 1)
    e5 = [((jrow // K == gcol) & (jrow % K == k)).astype(jnp.float32)
          for k in range(K)]
    for e in range(BS):
        counts = cnt_ref[e][:, :HIS]
        acc = None
        for k in range(K):
            tk = terms_ref[e, :, k, :]
            outk = jnp.dot(counts, tk, preferred_element_type=jnp.float32)
            part = jnp.dot(e5[k], outk, preferred_element_type=jnp.float32)
            acc = part if acc is None else acc + part
        out_ref[e] = acc


def kernel(ps_terms, ps_term_ids, his_news):
    B = ps_terms.shape[0]
    ids = ps_term_ids.reshape(B, N)
    ids_pad = jnp.pad(ids, ((0, 0), (0, 256 - N)))
    his2d = his_news.reshape(B, HIS * SL)

    counts = _counts_kernel(his2d, ids_pad).reshape(B, HIS, GCOLS)

    out = pl.pallas_call(
        _tc_body,
        grid=(B // BS,),
        in_specs=[
            pl.BlockSpec((BS, HIS, GCOLS), lambda i: (i, 0, 0)),
            pl.BlockSpec((BS, HIS, K, D), lambda i: (i, 0, 0, 0)),
        ],
        out_specs=pl.BlockSpec((BS, N, D), lambda i: (i, 0, 0)),
        out_shape=jax.ShapeDtypeStruct((B, N, D), jnp.float32),
    )(counts, ps_terms)
    return out


# speedup vs baseline: 46.1661x; 1.0560x over previous
import functools

import jax
import jax.numpy as jnp
from jax import lax
from jax.experimental import pallas as pl
from jax.experimental.pallas import tpu as pltpu
from jax.experimental.pallas import tpu_sc as plsc

HIS = 50
K = 5
N = HIS * K
D = 128
SL = 100
DK = K * D
VOCAB = 30522
TBL = 30528
GCOLS = 64
CLEN = HIS * GCOLS
NCHUNK = 16
BS = 4


def _lane_iota():
    return lax.broadcasted_iota(jnp.int32, (16,), 0)


def _take16(x, idx):
    dnums = lax.GatherDimensionNumbers(
        offset_dims=(), collapsed_slice_dims=(0,), start_index_map=(0,))
    return lax.gather(x, idx[:, None], dnums, (1,),
                      mode=lax.GatherScatterMode.PROMISE_IN_BOUNDS)


def _sc_body(his_hbm, ids_hbm, cnt_hbm, his_v, ids_v, vbuf, table, cflat):
    nc = 2
    wid = lax.axis_index("s") * nc + lax.axis_index("c")
    lanes = _lane_iota()
    zeros16 = jnp.zeros((16,), jnp.int32)
    zerosf = jnp.zeros((16,), jnp.float32)
    ones16 = jnp.ones((16,), jnp.float32)

    def _zt(k, _):
        table[pl.ds(k * 16, 16)] = zeros16
        return _
    lax.fori_loop(0, TBL // 16, _zt, 0)

    def _example(ex, _):
        i = wid * 8 + ex
        pltpu.sync_copy(his_hbm.at[i], his_v)
        pltpu.sync_copy(ids_hbm.at[i], ids_v)

        def _zc(k, _c):
            cflat[pl.ds(k * 16, 16)] = zerosf
            return _c
        lax.fori_loop(0, CLEN // 16, _zc, 0)

        for c in range(NCHUNK):
            jvec = c * 16 + lanes
            hvec = jnp.minimum(jvec // K, HIS - 1)
            tvec = ids_v[pl.ds(c * 16, 16)]
            addr = hvec * SL + tvec
            vbuf[pl.ds(c * 16, 16)] = plsc.load_gather(his_v, [addr])

        cnt = jnp.int32(0)
        for c in range(NCHUNK):
            jvec = c * 16 + lanes
            lanemask = jvec < N
            vvec = vbuf[pl.ds(c * 16, 16)]
            g16 = plsc.load_gather(table, [vvec])
            fpl = jnp.full((16,), 16, jnp.int32)
            for b in range(15, -1, -1):
                bv = _take16(vvec, jnp.full((16,), b, jnp.int32))
                fpl = jnp.where(vvec == bv, b, fpl)
            new_first = (fpl == lanes) & (g16 == 0) & lanemask
            rank = jnp.cumsum(new_first.astype(jnp.int32))
            gid_new = cnt + rank - 1
            gid16 = jnp.where(g16 > 0, g16 - 1, _take16(gid_new, fpl))
            plsc.store_scatter(table, [vvec], gid16 + 1, mask=lanemask)
            cnt = cnt + jnp.sum(new_first.astype(jnp.int32))
            cmask = lanemask & (gid16 < HIS)
            caddr = gid16 * GCOLS + ids_v[pl.ds(c * 16, 16)]
            plsc.addupdate_scatter(cflat, [caddr], ones16, mask=cmask)

        for c in range(NCHUNK):
            jvec = c * 16 + lanes
            vvec = vbuf[pl.ds(c * 16, 16)]
            plsc.store_scatter(table, [vvec], zeros16, mask=jvec < N)

        pltpu.sync_copy(cflat, cnt_hbm.at[i])
        return _
    lax.fori_loop(0, 8, _example, 0)


def _counts_kernel(his2d, ids2d):
    mesh = plsc.VectorSubcoreMesh(core_axis_name="c", subcore_axis_name="s")
    f = functools.partial(
        pl.kernel, mesh=mesh,
        compiler_params=pltpu.CompilerParams(needs_layout_passes=False),
        out_type=jax.ShapeDtypeStruct((256, CLEN), jnp.float32),
        scratch_types=[
            pltpu.VMEM((HIS * SL,), jnp.int32),
            pltpu.VMEM((256,), jnp.int32),
            pltpu.VMEM((256,), jnp.int32),
            pltpu.VMEM((TBL,), jnp.int32),
            pltpu.VMEM((CLEN,), jnp.float32),
        ],
    )(_sc_body)
    return f(his2d, ids2d)


def _tc_body(cnt_ref, terms_ref, out_ref):
    jrow = lax.broadcasted_iota(jnp.int32, (N, HIS), 0)
    gcol = lax.broadcasted_iota(jnp.int32, (N, HIS), 1)
    e5rep = (jrow // K == gcol).astype(jnp.float32)
    kmask = [(jrow % K == k) for k in range(K)]
    for e in range(BS):
        counts = cnt_ref[e][:, :HIS]
        crep = jnp.dot(e5rep, counts, preferred_element_type=jnp.float32)
        acc = None
        for k in range(K):
            ck = jnp.where(kmask[k], crep, 0.0)
            part = jnp.dot(ck, terms_ref[e, :, k, :],
                           preferred_element_type=jnp.float32)
            acc = part if acc is None else acc + part
        out_ref[e] = acc


def kernel(ps_terms, ps_term_ids, his_news):
    B = ps_terms.shape[0]
    ids = ps_term_ids.reshape(B, N)
    ids_pad = jnp.pad(ids, ((0, 0), (0, 256 - N)))
    his2d = his_news.reshape(B, HIS * SL)

    counts = _counts_kernel(his2d, ids_pad).reshape(B, HIS, GCOLS)

    out = pl.pallas_call(
        _tc_body,
        grid=(B // BS,),
        in_specs=[
            pl.BlockSpec((BS, HIS, GCOLS), lambda i: (i, 0, 0)),
            pl.BlockSpec((BS, HIS, K, D), lambda i: (i, 0, 0, 0)),
        ],
        out_specs=pl.BlockSpec((BS, N, D), lambda i: (i, 0, 0)),
        out_shape=jax.ShapeDtypeStruct((B, N, D), jnp.float32),
    )(counts, ps_terms)
    return out


# speedup vs baseline: 47.0934x; 1.0201x over previous
import functools

import jax
import jax.numpy as jnp
from jax import lax
from jax.experimental import pallas as pl
from jax.experimental.pallas import tpu as pltpu
from jax.experimental.pallas import tpu_sc as plsc

HIS = 50
K = 5
N = HIS * K
D = 128
SL = 100
DK = K * D
VOCAB = 30522
TBL = 30528
GCOLS = 64
CLEN = HIS * GCOLS
NCHUNK = 16
BS = 4


def _lane_iota():
    return lax.broadcasted_iota(jnp.int32, (16,), 0)


def _take16(x, idx):
    dnums = lax.GatherDimensionNumbers(
        offset_dims=(), collapsed_slice_dims=(0,), start_index_map=(0,))
    return lax.gather(x, idx[:, None], dnums, (1,),
                      mode=lax.GatherScatterMode.PROMISE_IN_BOUNDS)


def _sc_body(his_hbm, ids_hbm, cnt_hbm, his_v, ids_v, vbuf, table, cflat):
    nc = 2
    wid = lax.axis_index("s") * nc + lax.axis_index("c")
    lanes = _lane_iota()
    zeros16 = jnp.zeros((16,), jnp.int32)
    zerosf = jnp.zeros((16,), jnp.float32)
    ones16 = jnp.ones((16,), jnp.float32)

    def _zt(k, _):
        table[pl.ds(k * 16, 16)] = zeros16
        return _
    lax.fori_loop(0, TBL // 16, _zt, 0)

    def _example(ex, _):
        i = wid * 8 + ex
        pltpu.sync_copy(his_hbm.at[i], his_v)
        pltpu.sync_copy(ids_hbm.at[i], ids_v)

        def _zc(k, _c):
            cflat[pl.ds(k * 16, 16)] = zerosf
            return _c
        lax.fori_loop(0, CLEN // 16, _zc, 0)

        for c in range(NCHUNK):
            jvec = c * 16 + lanes
            hvec = jnp.minimum(jvec // K, HIS - 1)
            tvec = ids_v[pl.ds(c * 16, 16)]
            addr = hvec * SL + tvec
            vbuf[pl.ds(c * 16, 16)] = plsc.load_gather(his_v, [addr])

        cnt = jnp.int32(0)
        for c in range(NCHUNK):
            jvec = c * 16 + lanes
            lanemask = jvec < N
            vvec = vbuf[pl.ds(c * 16, 16)]
            g16 = plsc.load_gather(table, [vvec])
            fpl = jnp.full((16,), 16, jnp.int32)
            for b in range(15, -1, -1):
                bv = _take16(vvec, jnp.full((16,), b, jnp.int32))
                fpl = jnp.where(vvec == bv, b, fpl)
            new_first = (fpl == lanes) & (g16 == 0) & lanemask
            rank = jnp.cumsum(new_first.astype(jnp.int32))
            gid_new = cnt + rank - 1
            gid16 = jnp.where(g16 > 0, g16 - 1, _take16(gid_new, fpl))
            plsc.store_scatter(table, [vvec], gid16 + 1, mask=lanemask)
            cnt = cnt + jnp.sum(new_first.astype(jnp.int32))
            cmask = lanemask & (gid16 < HIS)
            caddr = gid16 * GCOLS + ids_v[pl.ds(c * 16, 16)]
            plsc.addupdate_scatter(cflat, [caddr], ones16, mask=cmask)

        for c in range(NCHUNK):
            jvec = c * 16 + lanes
            vvec = vbuf[pl.ds(c * 16, 16)]
            plsc.store_scatter(table, [vvec], zeros16, mask=jvec < N)

        pltpu.sync_copy(cflat, cnt_hbm.at[i])
        return _
    lax.fori_loop(0, 8, _example, 0)


def _counts_kernel(his2d, ids2d):
    mesh = plsc.VectorSubcoreMesh(core_axis_name="c", subcore_axis_name="s")
    f = functools.partial(
        pl.kernel, mesh=mesh,
        compiler_params=pltpu.CompilerParams(needs_layout_passes=False),
        out_type=jax.ShapeDtypeStruct((256, CLEN), jnp.float32),
        scratch_types=[
            pltpu.VMEM((HIS * SL,), jnp.int32),
            pltpu.VMEM((256,), jnp.int32),
            pltpu.VMEM((256,), jnp.int32),
            pltpu.VMEM((TBL,), jnp.int32),
            pltpu.VMEM((CLEN,), jnp.float32),
        ],
    )(_sc_body)
    return f(his2d, ids2d)


def _tc_body(cnt_ref, terms_ref, out_ref):
    jrow = lax.broadcasted_iota(jnp.int32, (N, HIS), 0)
    gcol = lax.broadcasted_iota(jnp.int32, (N, HIS), 1)
    e5rep = (jrow // K == gcol).astype(jnp.float32)
    kmask = [(jrow % K == k) for k in range(K)]
    for e in range(BS):
        counts = cnt_ref[e][:, :HIS]
        crep = jnp.dot(e5rep, counts, preferred_element_type=jnp.float32)
        t2 = terms_ref[e].reshape(HIS, DK)
        acc = None
        for k in range(K):
            ck = jnp.where(kmask[k], crep, 0.0)
            part = jnp.dot(ck, t2[:, k * D:(k + 1) * D],
                           preferred_element_type=jnp.float32)
            acc = part if acc is None else acc + part
        out_ref[e] = acc


def kernel(ps_terms, ps_term_ids, his_news):
    B = ps_terms.shape[0]
    ids = ps_term_ids.reshape(B, N)
    ids_pad = jnp.pad(ids, ((0, 0), (0, 256 - N)))
    his2d = his_news.reshape(B, HIS * SL)

    counts = _counts_kernel(his2d, ids_pad).reshape(B, HIS, GCOLS)

    out = pl.pallas_call(
        _tc_body,
        grid=(B // BS,),
        in_specs=[
            pl.BlockSpec((BS, HIS, GCOLS), lambda i: (i, 0, 0)),
            pl.BlockSpec((BS, HIS, K, D), lambda i: (i, 0, 0, 0)),
        ],
        out_specs=pl.BlockSpec((BS, N, D), lambda i: (i, 0, 0)),
        out_shape=jax.ShapeDtypeStruct((B, N, D), jnp.float32),
    )(counts, ps_terms)
    return out
